# baseline (speedup 1.0000x reference)
import functools

import jax
import jax.numpy as jnp
from jax import lax
from jax.experimental import pallas as pl
from jax.experimental.pallas import tpu as pltpu
from jax.experimental.pallas import tpu_sc as plsc

VOCAB = 100000
WORD_DIM = 300
DOMAIN_DIM = 100
OUT_DIM = WORD_DIM + DOMAIN_DIM
B = 1024
L = 200
N = B * L
NUM_WORKERS = 32
N_PER_W = N // NUM_WORKERS
CHUNK = 80
NCHUNKS = N_PER_W // CHUNK

FUSE_BLOCK = 1024


def _fuse_body(wt_ref, dt_ref, q0_ref, q1_ref, q2_ref, q3_ref):
    q0_ref[...] = wt_ref[pl.ds(0, 128), :].T
    q1_ref[...] = wt_ref[pl.ds(128, 128), :].T
    q2_ref[:, :44] = wt_ref[pl.ds(256, 44), :].T
    q2_ref[:, 44:] = dt_ref[pl.ds(0, 84), :].T
    q3_ref[...] = dt_ref[pl.ds(84, 16), :].T


_fuse_tables = pl.pallas_call(
    _fuse_body,
    grid=(pl.cdiv(VOCAB, FUSE_BLOCK),),
    in_specs=[
        pl.BlockSpec((WORD_DIM, FUSE_BLOCK), lambda i: (0, i)),
        pl.BlockSpec((DOMAIN_DIM, FUSE_BLOCK), lambda i: (0, i)),
    ],
    out_specs=[pl.BlockSpec((FUSE_BLOCK, 128), lambda i: (i, 0))] * 3 + [
        pl.BlockSpec((FUSE_BLOCK, 16), lambda i: (i, 0))],
    out_shape=[jax.ShapeDtypeStruct((VOCAB, 128), jnp.float32)] * 3 + [
        jax.ShapeDtypeStruct((VOCAB, 16), jnp.float32)],
)

_mesh = plsc.VectorSubcoreMesh(core_axis_name="c", subcore_axis_name="s")


def _make_gather(n):
    n_per_w = n // NUM_WORKERS
    nchunks = n_per_w // CHUNK

    @functools.partial(
        pl.kernel,
        mesh=_mesh,
        out_type=[jax.ShapeDtypeStruct((n, 128), jnp.float32)] * 3 + [
            jax.ShapeDtypeStruct((n, 16), jnp.float32)],
        compiler_params=pltpu.CompilerParams(use_tc_tiling_on_sc=False),
        scratch_types=[
            [pltpu.VMEM((CHUNK,), jnp.int32)] * 2,
            [[pltpu.VMEM((CHUNK, 128), jnp.float32)] * 3 +
             [pltpu.VMEM((CHUNK, 16), jnp.float32)]] * 2,
            [pltpu.SemaphoreType.DMA] * 2,
            [pltpu.SemaphoreType.DMA] * 2,
        ],
    )
    def gather_kernel(q0_hbm, q1_hbm, q2_hbm, q3_hbm, idx_hbm,
                      o0_hbm, o1_hbm, o2_hbm, o3_hbm,
                      idx_v, rows_v, gsem, ssem):
        wid = lax.axis_index("s") * 2 + lax.axis_index("c")
        base0 = wid * n_per_w
        tabs = (q0_hbm, q1_hbm, q2_hbm, q3_hbm)
        outs = (o0_hbm, o1_hbm, o2_hbm, o3_hbm)

        def gather_wait(i, slot):
            base = base0 + i * CHUNK
            pltpu.sync_copy(idx_hbm.at[pl.ds(base, CHUNK)], idx_v[slot])
            cps = [pltpu.async_copy(tabs[q].at[idx_v[slot]],
                                    rows_v[slot][q], gsem[slot])
                   for q in range(4)]
            for cp in cps:
                cp.wait()

        def put(i, slot):
            base = base0 + i * CHUNK
            for q in range(4):
                pltpu.async_copy(rows_v[slot][q],
                                 outs[q].at[pl.ds(base, CHUNK)], ssem[slot])

        def drain_put(slot):
            for q in range(4):
                pltpu.make_async_copy(rows_v[slot][q],
                                      outs[q].at[pl.ds(0, CHUNK)],
                                      ssem[slot]).wait()

        gather_wait(0, 0)
        put(0, 0)
        gather_wait(1, 1)

        def step(i, slot):
            put(i + 1, 1 - slot)
            drain_put(slot)
            gather_wait(i + 2, slot)

        def body(k, carry):
            step(2 * k, 0)
            step(2 * k + 1, 1)
            return carry

        lax.fori_loop(0, (nchunks - 2) // 2, body, 0)

        last = (nchunks - 1) % 2
        put(nchunks - 1, last)
        drain_put(1 - last)
        drain_put(last)

    return gather_kernel


_fused_gather_half = _make_gather(N // 2)


def _xpose_body(g0_ref, g1_ref, g2_ref, g3_ref, out_ref):
    out_ref[0, pl.ds(0, 128), :] = g0_ref[...].T
    out_ref[0, pl.ds(128, 128), :] = g1_ref[...].T
    out_ref[0, pl.ds(256, 128), :] = g2_ref[...].T
    out_ref[0, pl.ds(384, 16), :] = g3_ref[...].T


HALF = L // 2


def _make_xpose(half, aliased):
    def body(*refs):
        _xpose_body(*refs[-5:])

    n_in = 5 if aliased else 4
    in_specs = ([pl.BlockSpec((B, 128), lambda l: (l, 0))] * 3 +
                [pl.BlockSpec((B, 16), lambda l: (l, 0))])
    if aliased:
        in_specs = [pl.BlockSpec(memory_space=pl.ANY)] + in_specs
    return pl.pallas_call(
        body if aliased else _xpose_body,
        grid=(HALF,),
        in_specs=in_specs,
        out_specs=pl.BlockSpec((1, OUT_DIM, B),
                               lambda l, h=half: (l + h * HALF, 0, 0)),
        out_shape=jax.ShapeDtypeStruct((L, OUT_DIM, B), jnp.float32),
        input_output_aliases={0: 0} if aliased else {},
    )


_xpose_h0 = _make_xpose(0, aliased=False)
_xpose_h1 = _make_xpose(1, aliased=True)


def kernel(word_table, domain_table, x):
    q0, q1, q2, q3 = _fuse_tables(word_table.T, domain_table.T)
    idx = x.T.reshape(-1).astype(jnp.int32)
    halves = []
    for h in range(2):
        sl = slice(h * (N // 2), (h + 1) * (N // 2))
        halves.append(_fused_gather_half(q0, q1, q2, q3, idx[sl]))
    out3 = _xpose_h0(*halves[0])
    out3 = _xpose_h1(out3, *halves[1])
    return out3.transpose(2, 0, 1)


# speedup vs baseline: 1.1144x; 1.1144x over previous
import functools

import jax
import jax.numpy as jnp
from jax import lax
from jax.experimental import pallas as pl
from jax.experimental.pallas import tpu as pltpu
from jax.experimental.pallas import tpu_sc as plsc

VOCAB = 100000
WORD_DIM = 300
DOMAIN_DIM = 100
OUT_DIM = WORD_DIM + DOMAIN_DIM
B = 1024
L = 200
N = B * L
NUM_WORKERS = 32
N_PER_W = N // NUM_WORKERS
CHUNK = 80
NCHUNKS = N_PER_W // CHUNK

FUSE_BLOCK = 1024


def _fuse_body(wt_ref, dt_ref, q0_ref, q1_ref, q2_ref, q3_ref):
    q0_ref[...] = wt_ref[pl.ds(0, 128), :].T
    q1_ref[...] = wt_ref[pl.ds(128, 128), :].T
    q2_ref[:, :44] = wt_ref[pl.ds(256, 44), :].T
    q2_ref[:, 44:] = dt_ref[pl.ds(0, 84), :].T
    q3_ref[:, :16] = dt_ref[pl.ds(84, 16), :].T


_fuse_tables = pl.pallas_call(
    _fuse_body,
    grid=(pl.cdiv(VOCAB, FUSE_BLOCK),),
    in_specs=[
        pl.BlockSpec((WORD_DIM, FUSE_BLOCK), lambda i: (0, i)),
        pl.BlockSpec((DOMAIN_DIM, FUSE_BLOCK), lambda i: (0, i)),
    ],
    out_specs=[pl.BlockSpec((FUSE_BLOCK, 128), lambda i: (i, 0))] * 4,
    out_shape=[jax.ShapeDtypeStruct((VOCAB, 128), jnp.float32)] * 4,
)

_mesh = plsc.VectorSubcoreMesh(core_axis_name="c", subcore_axis_name="s")


def _make_gather(n):
    n_per_w = n // NUM_WORKERS
    nchunks = n_per_w // CHUNK

    @functools.partial(
        pl.kernel,
        mesh=_mesh,
        out_type=[jax.ShapeDtypeStruct((n, 128), jnp.float32)] * 4,
        compiler_params=pltpu.CompilerParams(use_tc_tiling_on_sc=False),
        scratch_types=[
            [pltpu.VMEM((CHUNK,), jnp.int32)] * 2,
            [[pltpu.VMEM((CHUNK, 128), jnp.float32)] * 4] * 2,
            [pltpu.SemaphoreType.DMA] * 2,
            [pltpu.SemaphoreType.DMA] * 2,
        ],
    )
    def gather_kernel(q0_hbm, q1_hbm, q2_hbm, q3_hbm, idx_hbm,
                      o0_hbm, o1_hbm, o2_hbm, o3_hbm,
                      idx_v, rows_v, gsem, ssem):
        wid = lax.axis_index("s") * 2 + lax.axis_index("c")
        base0 = wid * n_per_w
        tabs = (q0_hbm, q1_hbm, q2_hbm, q3_hbm)
        outs = (o0_hbm, o1_hbm, o2_hbm, o3_hbm)

        def gather_wait(i, slot):
            base = base0 + i * CHUNK
            pltpu.sync_copy(idx_hbm.at[pl.ds(base, CHUNK)], idx_v[slot])
            cps = [pltpu.async_copy(tabs[q].at[idx_v[slot]],
                                    rows_v[slot][q], gsem[slot])
                   for q in range(4)]
            for cp in cps:
                cp.wait()

        def put(i, slot):
            base = base0 + i * CHUNK
            for q in range(4):
                pltpu.async_copy(rows_v[slot][q],
                                 outs[q].at[pl.ds(base, CHUNK)], ssem[slot])

        def drain_put(slot):
            for q in range(4):
                pltpu.make_async_copy(rows_v[slot][q],
                                      outs[q].at[pl.ds(0, CHUNK)],
                                      ssem[slot]).wait()

        gather_wait(0, 0)
        put(0, 0)
        gather_wait(1, 1)

        def step(i, slot):
            put(i + 1, 1 - slot)
            drain_put(slot)
            gather_wait(i + 2, slot)

        def body(k, carry):
            step(2 * k, 0)
            step(2 * k + 1, 1)
            return carry

        lax.fori_loop(0, (nchunks - 2) // 2, body, 0)

        last = (nchunks - 1) % 2
        put(nchunks - 1, last)
        drain_put(1 - last)
        drain_put(last)

    return gather_kernel


_fused_gather_half = _make_gather(N // 2)


def _xpose_body(g0_ref, g1_ref, g2_ref, g3_ref, out_ref):
    out_ref[0, pl.ds(0, 128), :] = g0_ref[...].T
    out_ref[0, pl.ds(128, 128), :] = g1_ref[...].T
    out_ref[0, pl.ds(256, 128), :] = g2_ref[...].T
    out_ref[0, pl.ds(384, 16), :] = g3_ref[...].T


SLICES = 4
SLICE_L = L // SLICES


def _make_xpose(s, aliased):
    def body(*refs):
        _xpose_body(*refs[-5:])

    in_specs = ([pl.BlockSpec((B, 128), lambda l: (l, 0))] * 3 +
                [pl.BlockSpec((B, 16), lambda l: (l, 0))])
    if aliased:
        in_specs = [pl.BlockSpec(memory_space=pl.ANY)] + in_specs
    return pl.pallas_call(
        body if aliased else _xpose_body,
        grid=(SLICE_L,),
        in_specs=in_specs,
        out_specs=pl.BlockSpec((1, OUT_DIM, B),
                               lambda l, s=s: (l + s * SLICE_L, 0, 0)),
        out_shape=jax.ShapeDtypeStruct((L, OUT_DIM, B), jnp.float32),
        input_output_aliases={0: 0} if aliased else {},
    )


_xpose_slices = [_make_xpose(s, aliased=(s > 0)) for s in range(SLICES)]
_fused_gather_slice = _make_gather(N // SLICES)


def kernel(word_table, domain_table, x):
    q0, q1, q2, q3 = _fuse_tables(word_table.T, domain_table.T)
    idx = x.T.reshape(-1).astype(jnp.int32)
    ns = N // SLICES
    out3 = None
    for s in range(SLICES):
        g0, g1, g2, g3 = _fused_gather_slice(q0, q1, q2, q3,
                                             idx[s * ns:(s + 1) * ns])
        args = (g0, g1, g2, g3[:, :16])
        out3 = (_xpose_slices[0](*args) if s == 0
                else _xpose_slices[s](out3, *args))
    return out3.transpose(2, 0, 1)
